# baseline (speedup 1.0000x reference)
import functools

import jax
import jax.numpy as jnp
from jax import lax
from jax.experimental import pallas as pl
from jax.experimental.pallas import tpu as pltpu
from jax.experimental.pallas import tpu_sc as plsc

NW = 32
LANES = 16
CHUNK = 128
NCH = 2


def _make_route_scatter(n: int, d: int, n_exp: int, dtype):
    mesh = plsc.VectorSubcoreMesh(core_axis_name="c", subcore_axis_name="s")
    bpw = n // NW
    nv = bpw // LANES
    nck = bpw // CHUNK

    @functools.partial(
        pl.kernel,
        mesh=mesh,
        out_type=(
            jax.ShapeDtypeStruct((n, d), dtype),
            jax.ShapeDtypeStruct((n,), jnp.int32),
        ),
        scratch_types=[
            pltpu.VMEM((bpw,), jnp.int32),
            pltpu.VMEM((LANES,), jnp.int32),
            pltpu.VMEM((bpw,), jnp.int32),
            pltpu.VMEM((CHUNK,), jnp.int32),
            pltpu.VMEM((CHUNK,), jnp.int32),
            pltpu.VMEM((CHUNK, d), dtype),
            pltpu.SemaphoreType.DMA,
        ],
    )
    def route(x_hbm, d_hbm, base_hbm, xs_hbm, perm_hbm,
              d_v, off_v, tok_v, dc0, dc1, rows_v, sem):
        wid = lax.axis_index("s") * 2 + lax.axis_index("c")
        tok0 = wid * bpw
        pltpu.sync_copy(d_hbm.at[pl.ds(tok0, bpw)], d_v)
        pltpu.sync_copy(base_hbm.at[wid], off_v)
        lane = lax.iota(jnp.int32, LANES)
        off_vec = off_v[...]
        for k in range(nv):
            dk = d_v[pl.ds(k * LANES, LANES)]
            rank = jnp.zeros((LANES,), jnp.int32)
            cnt = jnp.zeros((LANES,), jnp.int32)
            for j in range(LANES):
                dj = dk.at[jnp.full((LANES,), j, jnp.int32)].get(
                    mode="promise_in_bounds")
                rank = rank + jnp.where((dj == dk) & (lane > j), 1, 0)
                cnt = cnt + jnp.where(lane == dj, 1, 0)
            base_pe = off_vec.at[dk].get(mode="promise_in_bounds")
            off_vec = off_vec + cnt
            dc = dc0 if k < CHUNK // LANES else dc1
            dc[pl.ds((k % (CHUNK // LANES)) * LANES, LANES)] = base_pe + rank
            tok_v[pl.ds(k * LANES, LANES)] = tok0 + k * LANES + lane
        for c, dc in zip(range(nck), (dc0, dc1)):
            pltpu.async_copy(tok_v.at[pl.ds(c * CHUNK, CHUNK)],
                             perm_hbm.at[dc], sem).wait()
            pltpu.sync_copy(x_hbm.at[pl.ds(tok0 + c * CHUNK, CHUNK)], rows_v)
            pltpu.async_copy(rows_v, xs_hbm.at[dc], sem).wait()

    return route


def _make_row_scatter(n_rows: int, d: int, dtype, n_chunks: int):
    mesh = plsc.VectorSubcoreMesh(core_axis_name="c", subcore_axis_name="s")
    bpw = n_rows // NW
    nch = bpw // CHUNK
    tiles_per_chunk = NW // n_chunks

    @functools.partial(
        pl.kernel,
        mesh=mesh,
        out_type=jax.ShapeDtypeStruct((n_rows, d), dtype),
        scratch_types=[
            pltpu.VMEM((CHUNK,), jnp.int32),
            pltpu.VMEM((CHUNK, d), dtype),
            pltpu.SemaphoreType.DMA,
        ],
    )
    def scatter(*args):
        srcs = args[:n_chunks]
        idx_hbm = args[n_chunks]
        out_hbm = args[n_chunks + 1]
        idx_v, rows_v, sem = args[n_chunks + 2:]
        wid = lax.axis_index("s") * 2 + lax.axis_index("c")
        for k in range(n_chunks):
            lo = k * tiles_per_chunk
            @pl.when((wid >= lo) & (wid < lo + tiles_per_chunk))
            def _():
                for c in range(nch):
                    base = wid * bpw + c * CHUNK
                    local = (wid - lo) * bpw + c * CHUNK
                    pltpu.sync_copy(idx_hbm.at[pl.ds(base, CHUNK)], idx_v)
                    pltpu.sync_copy(srcs[k].at[pl.ds(local, CHUNK)], rows_v)
                    pltpu.async_copy(rows_v, out_hbm.at[idx_v], sem).wait()

    return scatter


def _gmm_body(ends_ref, xs_ref, w_ref, b_ref, out_ref, *, block_rows, n_exp,
              row_base):
    i = pl.program_id(0)
    row0 = row_base + i * block_rows
    ridx = row0 + lax.broadcasted_iota(jnp.int32, (block_rows, 1), 0)
    e_row = jnp.zeros((block_rows, 1), jnp.int32)
    e_lo = jnp.int32(0)
    e_hi = jnp.int32(0)
    for e in range(n_exp - 1):
        end_e = ends_ref[e]
        e_row = e_row + (ridx >= end_e).astype(jnp.int32)
        e_lo = e_lo + (row0 >= end_e).astype(jnp.int32)
        e_hi = e_hi + (row0 + block_rows - 1 >= end_e).astype(jnp.int32)

    x = xs_ref[:]

    def body(e, _):
        y = jnp.dot(x, w_ref[e], preferred_element_type=jnp.float32)
        y = y + b_ref[e]
        out_ref[:] = jnp.where(e_row == e, y, out_ref[:])
        return 0

    out_ref[:] = jnp.zeros_like(out_ref)
    lax.fori_loop(e_lo, e_hi + 1, body, 0)


def _grouped_matmul(ends, xs, w, b3, block_rows: int, row_base: int,
                    rows: int):
    n, d_in = xs.shape
    n_exp, _, d_out = w.shape
    base_blk = row_base // block_rows
    grid = (rows // block_rows,)
    grid_spec = pltpu.PrefetchScalarGridSpec(
        num_scalar_prefetch=1,
        grid=grid,
        in_specs=[
            pl.BlockSpec((block_rows, d_in),
                         lambda i, ends: (base_blk + i, 0)),
            pl.BlockSpec((n_exp, d_in, d_out), lambda i, ends: (0, 0, 0)),
            pl.BlockSpec((n_exp, 1, d_out), lambda i, ends: (0, 0, 0)),
        ],
        out_specs=pl.BlockSpec((block_rows, d_out), lambda i, ends: (i, 0)),
    )
    return pl.pallas_call(
        functools.partial(_gmm_body, block_rows=block_rows, n_exp=n_exp,
                          row_base=row_base),
        grid_spec=grid_spec,
        out_shape=jax.ShapeDtypeStruct((rows, d_out), jnp.float32),
        compiler_params=pltpu.CompilerParams(
            dimension_semantics=("arbitrary",),
        ),
    )(ends, xs, w, b3)


def kernel(X, D, W, b):
    n, d_in = X.shape
    n_exp, _, d_out = W.shape
    rows_per_chunk = n // NCH
    bpw = n // NW

    d32 = D.reshape(NW, bpw).astype(jnp.int32)
    oh = (d32[:, :, None] == jnp.arange(n_exp, dtype=jnp.int32)).astype(jnp.int32)
    tile_cnt = jnp.sum(oh, axis=1)
    tot = jnp.sum(tile_cnt, axis=0)
    ends = jnp.cumsum(tot).astype(jnp.int32)
    starts = ends - tot
    tile_prefix = jnp.cumsum(tile_cnt, axis=0) - tile_cnt
    base = starts[None, :] + tile_prefix
    base16 = jnp.concatenate(
        [base, jnp.zeros((NW, LANES - n_exp), jnp.int32)], axis=1)

    route = _make_route_scatter(n, d_in, n_exp, X.dtype)
    xs, perm = route(X, D.astype(jnp.int32), base16)

    b3 = b.reshape(n_exp, 1, d_out)
    ys = [
        _grouped_matmul(ends, xs, W, b3, block_rows=256,
                        row_base=k * rows_per_chunk, rows=rows_per_chunk)
        for k in range(NCH)
    ]

    scatter = _make_row_scatter(n, d_out, jnp.float32, NCH)
    theta = scatter(*ys, perm)
    return theta


# speedup vs baseline: 1.2639x; 1.2639x over previous
import functools

import jax
import jax.numpy as jnp
from jax import lax
from jax.experimental import pallas as pl
from jax.experimental.pallas import tpu as pltpu
from jax.experimental.pallas import tpu_sc as plsc

NW = 32
LANES = 16
CHUNK = 128
NCH = 2


def _make_row_gather(n_rows: int, d: int, dtype):
    mesh = plsc.VectorSubcoreMesh(core_axis_name="c", subcore_axis_name="s")
    bpw = n_rows // NW
    ch = min(CHUNK, bpw)
    nch = bpw // ch

    @functools.partial(
        pl.kernel,
        mesh=mesh,
        out_type=jax.ShapeDtypeStruct((n_rows, d), dtype),
        scratch_types=[
            pltpu.VMEM((ch,), jnp.int32),
            pltpu.VMEM((ch, d), dtype),
            pltpu.SemaphoreType.DMA,
        ],
    )
    def gather(src_hbm, idx_hbm, out_hbm, idx_v, rows_v, sem):
        wid = lax.axis_index("s") * 2 + lax.axis_index("c")
        for c in range(nch):
            base = wid * bpw + c * ch
            pltpu.sync_copy(idx_hbm.at[pl.ds(base, ch)], idx_v)
            pltpu.async_copy(src_hbm.at[idx_v], rows_v, sem).wait()
            pltpu.sync_copy(rows_v, out_hbm.at[pl.ds(base, ch)])

    return gather


def _make_row_scatter(n_rows: int, d: int, dtype, n_chunks: int):
    mesh = plsc.VectorSubcoreMesh(core_axis_name="c", subcore_axis_name="s")
    bpw = n_rows // NW
    nch = bpw // CHUNK
    tiles_per_chunk = NW // n_chunks

    @functools.partial(
        pl.kernel,
        mesh=mesh,
        out_type=jax.ShapeDtypeStruct((n_rows, d), dtype),
        scratch_types=[
            pltpu.VMEM((CHUNK,), jnp.int32),
            pltpu.VMEM((CHUNK, d), dtype),
            pltpu.SemaphoreType.DMA,
        ],
    )
    def scatter(*args):
        srcs = args[:n_chunks]
        idx_hbm = args[n_chunks]
        out_hbm = args[n_chunks + 1]
        idx_v, rows_v, sem = args[n_chunks + 2:]
        wid = lax.axis_index("s") * 2 + lax.axis_index("c")
        for k in range(n_chunks):
            lo = k * tiles_per_chunk
            @pl.when((wid >= lo) & (wid < lo + tiles_per_chunk))
            def _():
                for c in range(nch):
                    base = wid * bpw + c * CHUNK
                    local = (wid - lo) * bpw + c * CHUNK
                    pltpu.sync_copy(idx_hbm.at[pl.ds(base, CHUNK)], idx_v)
                    pltpu.sync_copy(srcs[k].at[pl.ds(local, CHUNK)], rows_v)
                    pltpu.async_copy(rows_v, out_hbm.at[idx_v], sem).wait()

    return scatter


def _gmm_body(ends_ref, xs_ref, w_ref, b_ref, out_ref, *, block_rows, n_exp,
              row_base):
    i = pl.program_id(0)
    row0 = row_base + i * block_rows
    ridx = row0 + lax.broadcasted_iota(jnp.int32, (block_rows, 1), 0)
    e_row = jnp.zeros((block_rows, 1), jnp.int32)
    e_lo = jnp.int32(0)
    e_hi = jnp.int32(0)
    for e in range(n_exp - 1):
        end_e = ends_ref[e]
        e_row = e_row + (ridx >= end_e).astype(jnp.int32)
        e_lo = e_lo + (row0 >= end_e).astype(jnp.int32)
        e_hi = e_hi + (row0 + block_rows - 1 >= end_e).astype(jnp.int32)

    x = xs_ref[:]

    def body(e, _):
        y = jnp.dot(x, w_ref[e], preferred_element_type=jnp.float32)
        y = y + b_ref[e]
        out_ref[:] = jnp.where(e_row == e, y, out_ref[:])
        return 0

    out_ref[:] = jnp.zeros_like(out_ref)
    lax.fori_loop(e_lo, e_hi + 1, body, 0)


def _grouped_matmul(ends, xs, w, b3, block_rows: int, row_base: int):
    rows, d_in = xs.shape
    n_exp, _, d_out = w.shape
    grid = (rows // block_rows,)
    grid_spec = pltpu.PrefetchScalarGridSpec(
        num_scalar_prefetch=1,
        grid=grid,
        in_specs=[
            pl.BlockSpec((block_rows, d_in), lambda i, ends: (i, 0)),
            pl.BlockSpec((n_exp, d_in, d_out), lambda i, ends: (0, 0, 0)),
            pl.BlockSpec((n_exp, 1, d_out), lambda i, ends: (0, 0, 0)),
        ],
        out_specs=pl.BlockSpec((block_rows, d_out), lambda i, ends: (i, 0)),
    )
    return pl.pallas_call(
        functools.partial(_gmm_body, block_rows=block_rows, n_exp=n_exp,
                          row_base=row_base),
        grid_spec=grid_spec,
        out_shape=jax.ShapeDtypeStruct((rows, d_out), jnp.float32),
        compiler_params=pltpu.CompilerParams(
            dimension_semantics=("arbitrary",),
        ),
    )(ends, xs, w, b3)


def kernel(X, D, W, b):
    n, d_in = X.shape
    n_exp, _, d_out = W.shape
    rows_per_chunk = n // NCH

    key = D.astype(jnp.int32) * n + jnp.arange(n, dtype=jnp.int32)
    skey = jnp.sort(key)
    perm = skey % n
    ds_sorted = skey // n
    counts = jnp.sum(
        (ds_sorted[:, None] == jnp.arange(n_exp, dtype=jnp.int32)[None, :])
        .astype(jnp.int32), axis=0)
    ends = jnp.cumsum(counts).astype(jnp.int32)

    gather = _make_row_gather(rows_per_chunk, d_in, X.dtype)
    b3 = b.reshape(n_exp, 1, d_out)
    ys = []
    for k in range(NCH):
        perm_k = lax.dynamic_slice_in_dim(perm, k * rows_per_chunk,
                                          rows_per_chunk)
        xs_k = gather(X, perm_k)
        ys.append(_grouped_matmul(ends, xs_k, W, b3, block_rows=256,
                                  row_base=k * rows_per_chunk))

    scatter = _make_row_scatter(n, d_out, jnp.float32, NCH)
    theta = scatter(*ys, perm)
    return theta


# speedup vs baseline: 1.3397x; 1.0599x over previous
import functools

import jax
import jax.numpy as jnp
from jax import lax
from jax.experimental import pallas as pl
from jax.experimental.pallas import tpu as pltpu
from jax.experimental.pallas import tpu_sc as plsc

NW = 32
LANES = 16
CHUNK = 128
NCH = 2


def _make_row_gather(n_rows: int, d: int, dtype):
    mesh = plsc.VectorSubcoreMesh(core_axis_name="c", subcore_axis_name="s")
    bpw = n_rows // NW
    ch = min(CHUNK, bpw)
    nch = bpw // ch

    @functools.partial(
        pl.kernel,
        mesh=mesh,
        out_type=jax.ShapeDtypeStruct((n_rows, d), dtype),
        scratch_types=[
            pltpu.VMEM((ch,), jnp.int32),
            pltpu.VMEM((ch, d), dtype),
            pltpu.SemaphoreType.DMA,
        ],
    )
    def gather(src_hbm, idx_hbm, out_hbm, idx_v, rows_v, sem):
        wid = lax.axis_index("s") * 2 + lax.axis_index("c")
        for c in range(nch):
            base = wid * bpw + c * ch
            pltpu.sync_copy(idx_hbm.at[pl.ds(base, ch)], idx_v)
            pltpu.async_copy(src_hbm.at[idx_v], rows_v, sem).wait()
            pltpu.sync_copy(rows_v, out_hbm.at[pl.ds(base, ch)])

    return gather


def _make_row_scatter(n_rows: int, d: int, dtype, n_chunks: int):
    mesh = plsc.VectorSubcoreMesh(core_axis_name="c", subcore_axis_name="s")
    bpw = n_rows // NW
    nch = bpw // CHUNK
    tiles_per_chunk = NW // n_chunks

    @functools.partial(
        pl.kernel,
        mesh=mesh,
        out_type=jax.ShapeDtypeStruct((n_rows, d), dtype),
        scratch_types=[
            pltpu.VMEM((CHUNK,), jnp.int32),
            pltpu.VMEM((CHUNK, d), dtype),
            pltpu.SemaphoreType.DMA,
        ],
    )
    def scatter(*args):
        srcs = args[:n_chunks]
        idx_hbm = args[n_chunks]
        out_hbm = args[n_chunks + 1]
        idx_v, rows_v, sem = args[n_chunks + 2:]
        wid = lax.axis_index("s") * 2 + lax.axis_index("c")
        for k in range(n_chunks):
            lo = k * tiles_per_chunk
            @pl.when((wid >= lo) & (wid < lo + tiles_per_chunk))
            def _():
                for c in range(nch):
                    base = wid * bpw + c * CHUNK
                    local = (wid - lo) * bpw + c * CHUNK
                    pltpu.sync_copy(idx_hbm.at[pl.ds(base, CHUNK)], idx_v)
                    pltpu.sync_copy(srcs[k].at[pl.ds(local, CHUNK)], rows_v)
                    pltpu.async_copy(rows_v, out_hbm.at[idx_v], sem).wait()

    return scatter


def _gmm_body(ends_ref, xs_ref, w_ref, b_ref, out_ref, *, block_rows, n_exp,
              row_base):
    i = pl.program_id(0)
    row0 = row_base + i * block_rows
    ridx = row0 + lax.broadcasted_iota(jnp.int32, (block_rows, 1), 0)
    e_row = jnp.zeros((block_rows, 1), jnp.int32)
    e_lo = jnp.int32(0)
    e_hi = jnp.int32(0)
    for e in range(n_exp - 1):
        end_e = ends_ref[e]
        e_row = e_row + (ridx >= end_e).astype(jnp.int32)
        e_lo = e_lo + (row0 >= end_e).astype(jnp.int32)
        e_hi = e_hi + (row0 + block_rows - 1 >= end_e).astype(jnp.int32)

    x = xs_ref[:]

    def body(e, _):
        y = jnp.dot(x, w_ref[e], preferred_element_type=jnp.float32)
        y = y + b_ref[e]
        out_ref[:] = jnp.where(e_row == e, y, out_ref[:])
        return 0

    out_ref[:] = jnp.zeros_like(out_ref)
    lax.fori_loop(e_lo, e_hi + 1, body, 0)


def _grouped_matmul(ends, xs, w, b3, block_rows: int, row_base: int):
    rows, d_in = xs.shape
    n_exp, _, d_out = w.shape
    grid = (rows // block_rows,)
    grid_spec = pltpu.PrefetchScalarGridSpec(
        num_scalar_prefetch=1,
        grid=grid,
        in_specs=[
            pl.BlockSpec((block_rows, d_in), lambda i, ends: (i, 0)),
            pl.BlockSpec((n_exp, d_in, d_out), lambda i, ends: (0, 0, 0)),
            pl.BlockSpec((n_exp, 1, d_out), lambda i, ends: (0, 0, 0)),
        ],
        out_specs=pl.BlockSpec((block_rows, d_out), lambda i, ends: (i, 0)),
    )
    return pl.pallas_call(
        functools.partial(_gmm_body, block_rows=block_rows, n_exp=n_exp,
                          row_base=row_base),
        grid_spec=grid_spec,
        out_shape=jax.ShapeDtypeStruct((rows, d_out), jnp.float32),
        compiler_params=pltpu.CompilerParams(
            dimension_semantics=("arbitrary",),
        ),
    )(ends, xs, w, b3)


def kernel(X, D, W, b):
    n, d_in = X.shape
    n_exp, _, d_out = W.shape
    rows_per_chunk = n // NCH

    key = D.astype(jnp.int32) * n + jnp.arange(n, dtype=jnp.int32)
    skey = jnp.sort(key)
    perm = skey % n
    ends = jnp.searchsorted(
        skey, (jnp.arange(n_exp, dtype=jnp.int32) + 1) * n).astype(jnp.int32)

    gather = _make_row_gather(rows_per_chunk, d_in, X.dtype)
    b3 = b.reshape(n_exp, 1, d_out)
    ys = []
    for k in range(NCH):
        perm_k = lax.dynamic_slice_in_dim(perm, k * rows_per_chunk,
                                          rows_per_chunk)
        xs_k = gather(X, perm_k)
        ys.append(_grouped_matmul(ends, xs_k, W, b3, block_rows=512,
                                  row_base=k * rows_per_chunk))

    scatter = _make_row_scatter(n, d_out, jnp.float32, NCH)
    theta = scatter(*ys, perm)
    return theta
